# baseline (speedup 1.0000x reference)
import functools

import jax
import jax.numpy as jnp
import numpy as np
from jax import lax
from jax.experimental import pallas as pl
from jax.experimental.pallas import tpu as pltpu
from jax.experimental.pallas import tpu_sc as plsc

N = 10000
E = 320000
D = 128

NC = 2
NS = 16
CHUNK = 64
NBUF = 4
NSB = 2
NCH_A = 220
NCH_B = 96
NCHG = NS * (NCH_A + NCH_B)
EP = NCHG * CHUNK
SRC_PAD = (NCH_A - NCH_B) * CHUNK
ROWS_PER_TILE = 632
NP = ROWS_PER_TILE * NS

_PI = np.concatenate([
    np.concatenate([np.arange(g * 32, (g + 1) * 32, 2),
                    np.arange(g * 32 + 1, (g + 1) * 32, 2)])
    for g in range(D // 32)])

_SPLAT_DN = lax.GatherDimensionNumbers(
    offset_dims=(), collapsed_slice_dims=(0,), start_index_map=(0,))


def _splat(v, l):
    idx = jnp.full((16, 1), l, dtype=jnp.int32)
    return lax.gather(v, idx, _SPLAT_DN, (1,),
                      mode=lax.GatherScatterMode.PROMISE_IN_BOUNDS)


def _sc_body(hb_hbm, src_hbm, dst_hbm, w_hbm, zeros_hbm, out_hbm,
             src2d, dstb, wb, rowsb, rowsf, acc, *sems):
    gsem = sems[:NBUF]
    ssem = sems[NBUF:]
    ci = lax.axis_index("c")
    si = lax.axis_index("s")
    nch = jnp.where(ci == 0, NCH_A, NCH_B)
    nsup = nch // NBUF
    cbase = jnp.where(ci == 0, si * NCH_A, NS * NCH_A + si * NCH_B)

    def start_gather(c, b):
        idx = src2d.at[pl.ds(c * CHUNK, CHUNK)]
        e0 = (cbase + c) * CHUNK
        pltpu.async_copy(hb_hbm.at[idx], rowsb.at[b], gsem[b])
        pltpu.async_copy(dst_hbm.at[pl.ds(e0, CHUNK)], dstb.at[b], gsem[b])
        pltpu.async_copy(w_hbm.at[pl.ds(e0, CHUNK)], wb.at[b], gsem[b])

    def wait_gather(c, b):
        idx = src2d.at[pl.ds(c * CHUNK, CHUNK)]
        e0 = (cbase + c) * CHUNK
        pltpu.make_async_copy(hb_hbm.at[idx], rowsb.at[b], gsem[b]).wait()
        pltpu.make_async_copy(
            dst_hbm.at[pl.ds(e0, CHUNK)], dstb.at[b], gsem[b]).wait()
        pltpu.make_async_copy(
            w_hbm.at[pl.ds(e0, CHUNK)], wb.at[b], gsem[b]).wait()

    def start_scatter(b, s2):
        pltpu.async_copy(rowsf.at[s2], acc.at[dstb.at[b]], ssem[s2], add=True)

    def wait_scatter(b, s2):
        pltpu.make_async_copy(rowsf.at[s2], acc.at[dstb.at[b]], ssem[s2]).wait()

    pltpu.sync_copy(src_hbm.at[pl.ds(cbase * CHUNK, NCH_A * CHUNK)], src2d)
    pltpu.sync_copy(zeros_hbm, acc.at[pl.ds(si * ROWS_PER_TILE, ROWS_PER_TILE)])
    start_gather(0, 0)
    start_gather(1, 1)
    plsc.subcore_barrier()

    hi_mask = jnp.full((16,), -65536, dtype=jnp.int32)

    def super_body(s, _):
        for b in range(NBUF):
            c = s * NBUF + b
            s2 = b % NSB
            wait_gather(c, b)
            if b < 2:
                @pl.when(s >= 1)
                def _():
                    wait_scatter(b, s2)
            else:
                wait_scatter(b, s2)

            def group_body(g4, _):
                wv = wb[b, pl.ds(g4 * 16, 16)]
                for l in range(16):
                    wl = _splat(wv, l)
                    e = g4 * 16 + l
                    for g in range(D // 32):
                        xi = rowsb[b, e, pl.ds(g * 16, 16)]
                        lo = plsc.bitcast(xi << 16, jnp.float32)
                        hi = plsc.bitcast(xi & hi_mask, jnp.float32)
                        rowsf[s2, e, pl.ds(g * 32, 16)] = lo * wl
                        rowsf[s2, e, pl.ds(g * 32 + 16, 16)] = hi * wl
                return 0

            lax.fori_loop(0, CHUNK // 16, group_body, 0)
            start_scatter(b, s2)
            bp = (b + 2) % NBUF
            cp = c + 2
            if b < 2:
                start_gather(cp, bp)
            else:
                @pl.when(s < nsup - 1)
                def _():
                    start_gather(cp, bp)
        return 0

    lax.fori_loop(0, nsup, super_body, 0)
    wait_scatter(2, 0)
    wait_scatter(3, 1)
    plsc.subcore_barrier()
    sl = pl.ds(si * ROWS_PER_TILE, ROWS_PER_TILE)
    pltpu.sync_copy(acc.at[sl], out_hbm.at[ci, sl])


_sc_segment_sum = functools.partial(
    pl.kernel,
    out_type=jax.ShapeDtypeStruct((NC, NP, D), jnp.float32),
    mesh=plsc.VectorSubcoreMesh(
        core_axis_name="c", subcore_axis_name="s",
        num_cores=NC, num_subcores=NS),
    compiler_params=pltpu.CompilerParams(
        needs_layout_passes=False, use_tc_tiling_on_sc=False),
    scratch_types=[
        pltpu.VMEM((NCH_A * CHUNK,), jnp.int32),
        pltpu.VMEM((NBUF, CHUNK), jnp.int32),
        pltpu.VMEM((NBUF, CHUNK), jnp.float32),
        pltpu.VMEM((NBUF, CHUNK, D // 2), jnp.int32),
        pltpu.VMEM((NSB, CHUNK, D), jnp.float32),
        pltpu.VMEM_SHARED((NP, D), jnp.float32),
    ] + [pltpu.SemaphoreType.DMA] * (NBUF + NSB),
)(_sc_body)


def _gru_body(p_ref, h_ref, wih_ref, whh_ref, bih_ref, bhh_ref, out_ref):
    hn = p_ref[0] + p_ref[1]
    hb = h_ref[...]
    dn = (((1,), (1,)), ((), ()))
    gi = lax.dot_general(hn, wih_ref[...], dn,
                         preferred_element_type=jnp.float32) + bih_ref[...]
    gh = lax.dot_general(hb, whh_ref[...], dn,
                         preferred_element_type=jnp.float32) + bhh_ref[...]
    r = jax.nn.sigmoid(gi[:, :D] + gh[:, :D])
    z = jax.nn.sigmoid(gi[:, D:2 * D] + gh[:, D:2 * D])
    n = jnp.tanh(gi[:, 2 * D:] + r * gh[:, 2 * D:])
    out_ref[...] = (1.0 - z) * n + z * hb


def _gru(partials, h, W_ih, W_hh, b_ih, b_hh):
    B = 1000
    return pl.pallas_call(
        _gru_body,
        grid=(N // B,),
        in_specs=[
            pl.BlockSpec((NC, B, D), lambda i: (0, i, 0)),
            pl.BlockSpec((B, D), lambda i: (i, 0)),
            pl.BlockSpec((3 * D, D), lambda i: (0, 0)),
            pl.BlockSpec((3 * D, D), lambda i: (0, 0)),
            pl.BlockSpec((1, 3 * D), lambda i: (0, 0)),
            pl.BlockSpec((1, 3 * D), lambda i: (0, 0)),
        ],
        out_specs=pl.BlockSpec((B, D), lambda i: (i, 0)),
        out_shape=jax.ShapeDtypeStruct((N, D), jnp.float32),
    )(partials, h, W_ih, W_hh, b_ih, b_hh)


def kernel(h, edge_index, edge_weights, W_ih, W_hh, b_ih, b_hh):
    hb = jax.lax.bitcast_convert_type(
        h.astype(jnp.bfloat16).reshape(N, D // 2, 2), jnp.int32)
    src = jnp.pad(edge_index[0], (0, EP - E + SRC_PAD))
    dst = jnp.pad(edge_index[1], (0, EP - E))
    w = jnp.pad(edge_weights[:, 0], (0, EP - E))
    zeros = jnp.zeros((ROWS_PER_TILE, D), jnp.float32)
    partials = _sc_segment_sum(hb, src, dst, w, zeros)
    return _gru(partials, h, jnp.take(W_ih, _PI, axis=1), W_hh,
                b_ih.reshape(1, 3 * D), b_hh.reshape(1, 3 * D))


# speedup vs baseline: 1.5358x; 1.5358x over previous
import functools

import jax
import jax.numpy as jnp
from jax import lax
from jax.experimental import pallas as pl
from jax.experimental.pallas import tpu as pltpu
from jax.experimental.pallas import tpu_sc as plsc

N = 10000
E = 320000
D = 128

NC = 2
NS = 16
CHUNK = 64
NBUF = 4
NCH_A = 244
NCH_B = 72
NCHG = NS * (NCH_A + NCH_B)
EP = NCHG * CHUNK
SRC_PAD = (NCH_A - NCH_B) * CHUNK
ROWS_PER_TILE = 632
NP = ROWS_PER_TILE * NS

_SPLAT_DN = lax.GatherDimensionNumbers(
    offset_dims=(), collapsed_slice_dims=(0,), start_index_map=(0,))


def _splat(v, l):
    idx = jnp.full((16, 1), l, dtype=jnp.int32)
    return lax.gather(v, idx, _SPLAT_DN, (1,),
                      mode=lax.GatherScatterMode.PROMISE_IN_BOUNDS)


def _sc_body(h_hbm, src_hbm, dst_hbm, w_hbm, zeros_hbm, out_hbm,
             src2d, dstb, wb, rows, acc, *sems):
    gsem = sems[:NBUF]
    ssem = sems[NBUF:]
    ci = lax.axis_index("c")
    si = lax.axis_index("s")
    nch = jnp.where(ci == 0, NCH_A, NCH_B)
    nsup = nch // NBUF
    cbase = jnp.where(ci == 0, si * NCH_A, NS * NCH_A + si * NCH_B)

    def start_gather(c, b):
        idx = src2d.at[pl.ds(c * CHUNK, CHUNK)]
        e0 = (cbase + c) * CHUNK
        pltpu.async_copy(h_hbm.at[idx], rows.at[b], gsem[b])
        pltpu.async_copy(dst_hbm.at[pl.ds(e0, CHUNK)], dstb.at[b], gsem[b])
        pltpu.async_copy(w_hbm.at[pl.ds(e0, CHUNK)], wb.at[b], gsem[b])

    def wait_gather(c, b):
        idx = src2d.at[pl.ds(c * CHUNK, CHUNK)]
        e0 = (cbase + c) * CHUNK
        pltpu.make_async_copy(h_hbm.at[idx], rows.at[b], gsem[b]).wait()
        pltpu.make_async_copy(
            dst_hbm.at[pl.ds(e0, CHUNK)], dstb.at[b], gsem[b]).wait()
        pltpu.make_async_copy(
            w_hbm.at[pl.ds(e0, CHUNK)], wb.at[b], gsem[b]).wait()

    def start_scatter(b):
        pltpu.async_copy(rows.at[b], acc.at[dstb.at[b]], ssem[b], add=True)

    def wait_scatter(b):
        pltpu.make_async_copy(rows.at[b], acc.at[dstb.at[b]], ssem[b]).wait()

    pltpu.sync_copy(src_hbm.at[pl.ds(cbase * CHUNK, NCH_A * CHUNK)], src2d)
    pltpu.sync_copy(zeros_hbm, acc.at[pl.ds(si * ROWS_PER_TILE, ROWS_PER_TILE)])
    start_gather(0, 0)
    start_gather(1, 1)
    plsc.subcore_barrier()

    def super_body(s, _):
        for b in range(NBUF):
            c = s * NBUF + b
            wait_gather(c, b)

            def group_body(g, _):
                wv = wb[b, pl.ds(g * 16, 16)]
                for l in range(16):
                    wl = _splat(wv, l)
                    e = g * 16 + l
                    for j in range(D // 16):
                        sl = pl.ds(j * 16, 16)
                        rows[b, e, sl] = rows[b, e, sl] * wl
                return 0

            lax.fori_loop(0, CHUNK // 16, group_body, 0)
            start_scatter(b)
            bp = (b + 2) % NBUF
            cp = c + 2
            if b < 2:
                @pl.when(s >= 1)
                def _():
                    wait_scatter(bp)
                    start_gather(cp, bp)

                @pl.when(s == 0)
                def _():
                    start_gather(cp, bp)
            else:
                @pl.when(s < nsup - 1)
                def _():
                    wait_scatter(bp)
                    start_gather(cp, bp)
        return 0

    lax.fori_loop(0, nsup, super_body, 0)
    for b in range(NBUF):
        wait_scatter(b)
    plsc.subcore_barrier()
    sl = pl.ds(si * ROWS_PER_TILE, ROWS_PER_TILE)
    pltpu.sync_copy(acc.at[sl], out_hbm.at[ci, sl])


_sc_segment_sum = functools.partial(
    pl.kernel,
    out_type=jax.ShapeDtypeStruct((NC, NP, D), jnp.float32),
    mesh=plsc.VectorSubcoreMesh(
        core_axis_name="c", subcore_axis_name="s",
        num_cores=NC, num_subcores=NS),
    scratch_types=[
        pltpu.VMEM((NCH_A * CHUNK,), jnp.int32),
        pltpu.VMEM((NBUF, CHUNK), jnp.int32),
        pltpu.VMEM((NBUF, CHUNK), jnp.float32),
        pltpu.VMEM((NBUF, CHUNK, D), jnp.float32),
        pltpu.VMEM_SHARED((NP, D), jnp.float32),
    ] + [pltpu.SemaphoreType.DMA] * (2 * NBUF),
)(_sc_body)


def _gh_body(h_ref, whh_ref, bhh_ref, out_ref):
    dn = (((1,), (1,)), ((), ()))
    out_ref[...] = lax.dot_general(
        h_ref[...], whh_ref[...], dn,
        preferred_element_type=jnp.float32) + bhh_ref[...]


def _gh(h, W_hh, b_hh):
    B = 1000
    return pl.pallas_call(
        _gh_body,
        grid=(N // B,),
        in_specs=[
            pl.BlockSpec((B, D), lambda i: (i, 0)),
            pl.BlockSpec((3 * D, D), lambda i: (0, 0)),
            pl.BlockSpec((1, 3 * D), lambda i: (0, 0)),
        ],
        out_specs=pl.BlockSpec((B, 3 * D), lambda i: (i, 0)),
        out_shape=jax.ShapeDtypeStruct((N, 3 * D), jnp.float32),
    )(h, W_hh, b_hh)


def _gru_body(p_ref, h_ref, gh_ref, wih_ref, bih_ref, out_ref):
    hn = p_ref[0] + p_ref[1]
    hb = h_ref[...]
    gh = gh_ref[...]
    dn = (((1,), (1,)), ((), ()))
    gi = lax.dot_general(hn, wih_ref[...], dn,
                         preferred_element_type=jnp.float32) + bih_ref[...]
    r = jax.nn.sigmoid(gi[:, :D] + gh[:, :D])
    z = jax.nn.sigmoid(gi[:, D:2 * D] + gh[:, D:2 * D])
    n = jnp.tanh(gi[:, 2 * D:] + r * gh[:, 2 * D:])
    out_ref[...] = (1.0 - z) * n + z * hb


def _gru(partials, h, gh, W_ih, b_ih):
    B = 1000
    return pl.pallas_call(
        _gru_body,
        grid=(N // B,),
        in_specs=[
            pl.BlockSpec((NC, B, D), lambda i: (0, i, 0)),
            pl.BlockSpec((B, D), lambda i: (i, 0)),
            pl.BlockSpec((B, 3 * D), lambda i: (i, 0)),
            pl.BlockSpec((3 * D, D), lambda i: (0, 0)),
            pl.BlockSpec((1, 3 * D), lambda i: (0, 0)),
        ],
        out_specs=pl.BlockSpec((B, D), lambda i: (i, 0)),
        out_shape=jax.ShapeDtypeStruct((N, D), jnp.float32),
    )(partials, h, gh, W_ih, b_ih)


def kernel(h, edge_index, edge_weights, W_ih, W_hh, b_ih, b_hh):
    src = jnp.pad(edge_index[0], (0, EP - E + SRC_PAD))
    dst = jnp.pad(edge_index[1], (0, EP - E))
    w = jnp.pad(edge_weights[:, 0], (0, EP - E))
    zeros = jnp.zeros((ROWS_PER_TILE, D), jnp.float32)
    gh = _gh(h, W_hh, b_hh.reshape(1, 3 * D))
    partials = _sc_segment_sum(h, src, dst, w, zeros)
    return _gru(partials, h, gh, W_ih, b_ih.reshape(1, 3 * D))


# speedup vs baseline: 1.5665x; 1.0200x over previous
import functools

import jax
import jax.numpy as jnp
from jax import lax
from jax.experimental import pallas as pl
from jax.experimental.pallas import tpu as pltpu
from jax.experimental.pallas import tpu_sc as plsc

N = 10000
E = 320000
D = 128

NC = 2
NS = 16
CHUNK = 64
NBUF = 4
NCH_A = 244
NCH_B = 72
NCHG = NS * (NCH_A + NCH_B)
EP = NCHG * CHUNK
SRC_PAD = (NCH_A - NCH_B) * CHUNK
ROWS_PER_TILE = 632
NP = ROWS_PER_TILE * NS

_SPLAT_DN = lax.GatherDimensionNumbers(
    offset_dims=(), collapsed_slice_dims=(0,), start_index_map=(0,))


def _splat(v, l):
    idx = jnp.full((16, 1), l, dtype=jnp.int32)
    return lax.gather(v, idx, _SPLAT_DN, (1,),
                      mode=lax.GatherScatterMode.PROMISE_IN_BOUNDS)


def _sc_body(h_hbm, src_hbm, dst_hbm, w_hbm, zeros_hbm, out_hbm,
             src2d, dstb, wb, rows, acc, *sems):
    gsem = sems[:NBUF]
    ssem = sems[NBUF:]
    ci = lax.axis_index("c")
    si = lax.axis_index("s")
    nch = jnp.where(ci == 0, NCH_A, NCH_B)
    nsup = nch // NBUF
    cbase = jnp.where(ci == 0, si * NCH_A, NS * NCH_A + si * NCH_B)

    def start_gather(c, b):
        idx = src2d.at[pl.ds(c * CHUNK, CHUNK)]
        e0 = (cbase + c) * CHUNK
        pltpu.async_copy(h_hbm.at[idx], rows.at[b], gsem[b])
        pltpu.async_copy(dst_hbm.at[pl.ds(e0, CHUNK)], dstb.at[b], gsem[b])
        pltpu.async_copy(w_hbm.at[pl.ds(e0, CHUNK)], wb.at[b], gsem[b])

    def wait_gather(c, b):
        idx = src2d.at[pl.ds(c * CHUNK, CHUNK)]
        e0 = (cbase + c) * CHUNK
        pltpu.make_async_copy(h_hbm.at[idx], rows.at[b], gsem[b]).wait()
        pltpu.make_async_copy(
            dst_hbm.at[pl.ds(e0, CHUNK)], dstb.at[b], gsem[b]).wait()
        pltpu.make_async_copy(
            w_hbm.at[pl.ds(e0, CHUNK)], wb.at[b], gsem[b]).wait()

    def start_scatter(b):
        pltpu.async_copy(rows.at[b], acc.at[dstb.at[b]], ssem[b], add=True)

    def wait_scatter(b):
        pltpu.make_async_copy(rows.at[b], acc.at[dstb.at[b]], ssem[b]).wait()

    pltpu.sync_copy(src_hbm.at[pl.ds(cbase * CHUNK, NCH_A * CHUNK)], src2d)
    pltpu.sync_copy(zeros_hbm, acc.at[pl.ds(si * ROWS_PER_TILE, ROWS_PER_TILE)])
    start_gather(0, 0)
    start_gather(1, 1)
    plsc.subcore_barrier()

    def super_body(s, _):
        for b in range(NBUF):
            c = s * NBUF + b
            wait_gather(c, b)

            def group_body(g, _):
                wv = wb[b, pl.ds(g * 16, 16)]
                for l in range(16):
                    wl = _splat(wv, l)
                    e = g * 16 + l
                    for j in range(D // 16):
                        sl = pl.ds(j * 16, 16)
                        rows[b, e, sl] = rows[b, e, sl] * wl
                return 0

            lax.fori_loop(0, CHUNK // 16, group_body, 0)
            start_scatter(b)
            bp = (b + 2) % NBUF
            cp = c + 2
            if b < 2:
                @pl.when(s >= 1)
                def _():
                    wait_scatter(bp)
                    start_gather(cp, bp)

                @pl.when(s == 0)
                def _():
                    start_gather(cp, bp)
            else:
                @pl.when(s < nsup - 1)
                def _():
                    wait_scatter(bp)
                    start_gather(cp, bp)
        return 0

    lax.fori_loop(0, nsup, super_body, 0)
    for b in range(NBUF):
        wait_scatter(b)
    plsc.subcore_barrier()
    sl = pl.ds(si * ROWS_PER_TILE, ROWS_PER_TILE)
    pltpu.sync_copy(acc.at[sl], out_hbm.at[ci, sl])


_sc_segment_sum = functools.partial(
    pl.kernel,
    out_type=jax.ShapeDtypeStruct((NC, NP, D), jnp.float32),
    mesh=plsc.VectorSubcoreMesh(
        core_axis_name="c", subcore_axis_name="s",
        num_cores=NC, num_subcores=NS),
    scratch_types=[
        pltpu.VMEM((NCH_A * CHUNK,), jnp.int32),
        pltpu.VMEM((NBUF, CHUNK), jnp.int32),
        pltpu.VMEM((NBUF, CHUNK), jnp.float32),
        pltpu.VMEM((NBUF, CHUNK, D), jnp.float32),
        pltpu.VMEM_SHARED((NP, D), jnp.float32),
    ] + [pltpu.SemaphoreType.DMA] * (2 * NBUF),
)(_sc_body)


def _gru_body(p_ref, h_ref, wih_ref, whh_ref, bih_ref, bhh_ref, out_ref):
    hn = p_ref[0] + p_ref[1]
    hb = h_ref[...]
    dn = (((1,), (1,)), ((), ()))
    gi = lax.dot_general(hn, wih_ref[...], dn,
                         preferred_element_type=jnp.float32) + bih_ref[...]
    gh = lax.dot_general(hb, whh_ref[...], dn,
                         preferred_element_type=jnp.float32) + bhh_ref[...]
    r = jax.nn.sigmoid(gi[:, :D] + gh[:, :D])
    z = jax.nn.sigmoid(gi[:, D:2 * D] + gh[:, D:2 * D])
    n = jnp.tanh(gi[:, 2 * D:] + r * gh[:, 2 * D:])
    out_ref[...] = (1.0 - z) * n + z * hb


def _gru(partials, h, W_ih, W_hh, b_ih, b_hh):
    B = 1000
    return pl.pallas_call(
        _gru_body,
        grid=(N // B,),
        in_specs=[
            pl.BlockSpec((NC, B, D), lambda i: (0, i, 0)),
            pl.BlockSpec((B, D), lambda i: (i, 0)),
            pl.BlockSpec((3 * D, D), lambda i: (0, 0)),
            pl.BlockSpec((3 * D, D), lambda i: (0, 0)),
            pl.BlockSpec((1, 3 * D), lambda i: (0, 0)),
            pl.BlockSpec((1, 3 * D), lambda i: (0, 0)),
        ],
        out_specs=pl.BlockSpec((B, D), lambda i: (i, 0)),
        out_shape=jax.ShapeDtypeStruct((N, D), jnp.float32),
    )(partials, h, W_ih, W_hh, b_ih, b_hh)


def kernel(h, edge_index, edge_weights, W_ih, W_hh, b_ih, b_hh):
    src = jnp.pad(edge_index[0], (0, EP - E + SRC_PAD))
    dst = jnp.pad(edge_index[1], (0, EP - E))
    w = jnp.pad(edge_weights[:, 0], (0, EP - E))
    zeros = jnp.zeros((ROWS_PER_TILE, D), jnp.float32)
    partials = _sc_segment_sum(h, src, dst, w, zeros)
    return _gru(partials, h, W_ih, W_hh,
                b_ih.reshape(1, 3 * D), b_hh.reshape(1, 3 * D))
